# baseline (speedup 1.0000x reference)
import functools

import jax
import jax.numpy as jnp
from jax import lax
from jax.experimental import pallas as pl
from jax.experimental.pallas import tpu as pltpu
from jax.experimental.pallas import tpu_sc as plsc

L = 16
PP = 32
JP = 20
SLOTS = JP * PP
GCH = 16
GW = SLOTS // GCH


def _bc(s):
    return jnp.broadcast_to(s, (L,))


@functools.lru_cache(maxsize=None)
def _build(B, N, P, J):
    mesh = plsc.VectorSubcoreMesh(core_axis_name="c", subcore_axis_name="s")
    NC = 2

    @functools.partial(
        pl.kernel,
        mesh=mesh,
        out_type=jax.ShapeDtypeStruct((B, L), jnp.float32),
        compiler_params=pltpu.CompilerParams(needs_layout_passes=False),
        scratch_types=[
            pltpu.VMEM((GCH, GW), jnp.int32),
            pltpu.VMEM((SLOTS,), jnp.int32),
            pltpu.VMEM((SLOTS,), jnp.float32),
            pltpu.VMEM((L,), jnp.float32),
            pltpu.SemaphoreType.DMA,
        ],
    )
    def aeloss(tags_hbm, gidx_hbm, flg_hbm, out_hbm, gidx_v, flg_v, val_v,
               oval_v, sem):
        wid = lax.axis_index("s") * NC + lax.axis_index("c")
        zero = jnp.zeros((L,), jnp.float32)
        one = jnp.full((L,), 1.0, jnp.float32)
        lane = lax.iota(jnp.int32, L)

        with jax.named_scope("stage"):
            pltpu.sync_copy(gidx_hbm.at[wid], gidx_v)
        with jax.named_scope("fire"):
            copies = [
                pltpu.async_copy(
                    tags_hbm.at[gidx_v.at[r]], val_v.at[pl.ds(r * GW, GW)], sem
                )
                for r in range(GCH)
            ]
        with jax.named_scope("stage2"):
            pltpu.sync_copy(flg_hbm.at[wid], flg_v)
        with jax.named_scope("drain"):
            for cp in copies:
                cp.wait()

        scope_a = jax.named_scope("passA")
        scope_a.__enter__()

        def body_a(j, carry):
            cnt_lo, cnt_hi, sum_lo, sum_hi = carry
            base = j * PP
            f_lo = flg_v[pl.ds(base, L)]
            f_hi = flg_v[pl.ds(base + L, L)]
            v_lo = val_v[pl.ds(base, L)]
            v_hi = val_v[pl.ds(base + L, L)]
            w_lo = jnp.where(f_lo > 0, one, zero)
            w_hi = jnp.where(f_hi > 0, one, zero)
            return (cnt_lo + w_lo, cnt_hi + w_hi,
                    sum_lo + v_lo * w_lo, sum_hi + v_hi * w_hi)

        cnt_lo, cnt_hi, sum_lo, sum_hi = lax.fori_loop(
            0, J, body_a, (zero, zero, zero, zero))
        safe_lo = jnp.maximum(cnt_lo, one)
        safe_hi = jnp.maximum(cnt_hi, one)
        mean_lo = sum_lo / safe_lo
        mean_hi = sum_hi / safe_hi
        valid_lo = jnp.where(cnt_lo > 0, one, zero)
        valid_hi = jnp.where(cnt_hi > 0, one, zero)
        scope_a.__exit__(None, None, None)

        scope_b = jax.named_scope("passB")
        scope_b.__enter__()

        def body_b(j, carry):
            pacc_lo, pacc_hi = carry
            base = j * PP
            f_lo = flg_v[pl.ds(base, L)]
            f_hi = flg_v[pl.ds(base + L, L)]
            w_lo = jnp.where(f_lo > 0, one, zero)
            w_hi = jnp.where(f_hi > 0, one, zero)
            d_lo = val_v[pl.ds(base, L)] - mean_lo
            d_hi = val_v[pl.ds(base + L, L)] - mean_hi
            return (pacc_lo + d_lo * d_lo * w_lo, pacc_hi + d_hi * d_hi * w_hi)

        pacc_lo, pacc_hi = lax.fori_loop(0, J, body_b, (zero, zero))
        pull_s = jnp.sum(pacc_lo / safe_lo * valid_lo) + jnp.sum(
            pacc_hi / safe_hi * valid_hi)
        ntags = _bc(jnp.sum(valid_lo) + jnp.sum(valid_hi))
        scope_b.__exit__(None, None, None)

        scope_p = jax.named_scope("push")
        scope_p.__enter__()

        def push_pairs(src_m, src_v, n):
            def body_p(i, carry):
                acc_lo, acc_hi = carry
                sel = lane == i
                m_i = _bc(jnp.sum(jnp.where(sel, src_m, zero)))
                v_i = _bc(jnp.sum(jnp.where(sel, src_v, zero)))
                d_lo = m_i - mean_lo
                d_hi = m_i - mean_hi
                return (acc_lo + v_i * jnp.exp(-(d_lo * d_lo)) * valid_lo,
                        acc_hi + v_i * jnp.exp(-(d_hi * d_hi)) * valid_hi)

            return lax.fori_loop(0, n, body_p, (zero, zero))

        a_lo, a_hi = push_pairs(mean_lo, valid_lo, L)
        b_lo, b_hi = push_pairs(mean_hi, valid_hi, P - L)
        acc_lo = a_lo + b_lo
        acc_hi = a_hi + b_hi
        push_tot = _bc(jnp.sum(acc_lo) + jnp.sum(acc_hi)) - ntags
        denom = jnp.maximum(ntags * (ntags - one), one)
        push = 0.5 * push_tot / denom
        pull = _bc(pull_s) / jnp.maximum(ntags, one)
        scope_p.__exit__(None, None, None)

        with jax.named_scope("out"):
            oval_v[...] = jnp.where(
                lane == 0, pull, jnp.where(lane == 1, push, zero))
            pltpu.sync_copy(oval_v, out_hbm.at[wid])

    return aeloss


def kernel(input, input1):
    tags = input
    keypoints = input1
    B, N, D = tags.shape
    P, J = keypoints.shape[1], keypoints.shape[2]

    idx_t = keypoints[..., 0].transpose(0, 2, 1)
    flg_t = keypoints[..., 1].transpose(0, 2, 1)
    gidx = (jnp.zeros((B, JP, PP), jnp.int32).at[:, :J, :P].set(idx_t)
            + (jnp.arange(B, dtype=jnp.int32) * N)[:, None, None])
    flgp = jnp.zeros((B, JP, PP), jnp.int32).at[:, :J, :P].set(flg_t)

    out = _build(B, N, P, J)(
        tags.reshape(B * N),
        gidx.reshape(B, GCH, GW),
        flgp.reshape(B, SLOTS),
    )
    return out[:, :2]


# speedup vs baseline: 1.1443x; 1.1443x over previous
import functools

import jax
import jax.numpy as jnp
from jax import lax
from jax.experimental import pallas as pl
from jax.experimental.pallas import tpu as pltpu
from jax.experimental.pallas import tpu_sc as plsc

L = 16
PP = 32
JP = 20
SLOTS = JP * PP
GCH = 16
GW = SLOTS // GCH


def _bc(s):
    return jnp.broadcast_to(s, (L,))


@functools.lru_cache(maxsize=None)
def _build(B, N, P, J):
    mesh = plsc.VectorSubcoreMesh(core_axis_name="c", subcore_axis_name="s")
    NC = 2

    @functools.partial(
        pl.kernel,
        mesh=mesh,
        out_type=jax.ShapeDtypeStruct((B, L), jnp.float32),
        compiler_params=pltpu.CompilerParams(needs_layout_passes=False),
        scratch_types=[
            pltpu.VMEM((GCH, GW), jnp.int32),
            pltpu.VMEM((SLOTS,), jnp.int32),
            pltpu.VMEM((SLOTS,), jnp.float32),
            pltpu.VMEM((L,), jnp.float32),
            pltpu.SemaphoreType.DMA,
        ],
    )
    def aeloss(tags_hbm, gidx_hbm, flg_hbm, out_hbm, gidx_v, flg_v, val_v,
               oval_v, sem):
        wid = lax.axis_index("s") * NC + lax.axis_index("c")
        zero = jnp.zeros((L,), jnp.float32)
        one = jnp.full((L,), 1.0, jnp.float32)
        lane = lax.iota(jnp.int32, L)

        with jax.named_scope("stage"):
            pltpu.sync_copy(gidx_hbm.at[wid], gidx_v)
        with jax.named_scope("fire"):
            copies = [
                pltpu.async_copy(
                    tags_hbm.at[gidx_v.at[r]], val_v.at[pl.ds(r * GW, GW)], sem
                )
                for r in range(GCH)
            ]
        with jax.named_scope("stage2"):
            pltpu.sync_copy(flg_hbm.at[wid], flg_v)
        with jax.named_scope("drain"):
            for cp in copies:
                cp.wait()

        scope_a = jax.named_scope("passA")
        scope_a.__enter__()

        def body_a(j, carry):
            cnt_lo, cnt_hi, sum_lo, sum_hi = carry
            base = j * PP
            f_lo = flg_v[pl.ds(base, L)]
            f_hi = flg_v[pl.ds(base + L, L)]
            v_lo = val_v[pl.ds(base, L)]
            v_hi = val_v[pl.ds(base + L, L)]
            w_lo = jnp.where(f_lo > 0, one, zero)
            w_hi = jnp.where(f_hi > 0, one, zero)
            return (cnt_lo + w_lo, cnt_hi + w_hi,
                    sum_lo + v_lo * w_lo, sum_hi + v_hi * w_hi)

        cnt_lo, cnt_hi, sum_lo, sum_hi = lax.fori_loop(
            0, J, body_a, (zero, zero, zero, zero))
        safe_lo = jnp.maximum(cnt_lo, one)
        safe_hi = jnp.maximum(cnt_hi, one)
        mean_lo = sum_lo / safe_lo
        mean_hi = sum_hi / safe_hi
        valid_lo = jnp.where(cnt_lo > 0, one, zero)
        valid_hi = jnp.where(cnt_hi > 0, one, zero)
        scope_a.__exit__(None, None, None)

        scope_b = jax.named_scope("passB")
        scope_b.__enter__()

        def body_b(j, carry):
            pacc_lo, pacc_hi = carry
            base = j * PP
            f_lo = flg_v[pl.ds(base, L)]
            f_hi = flg_v[pl.ds(base + L, L)]
            w_lo = jnp.where(f_lo > 0, one, zero)
            w_hi = jnp.where(f_hi > 0, one, zero)
            d_lo = val_v[pl.ds(base, L)] - mean_lo
            d_hi = val_v[pl.ds(base + L, L)] - mean_hi
            return (pacc_lo + d_lo * d_lo * w_lo, pacc_hi + d_hi * d_hi * w_hi)

        pacc_lo, pacc_hi = lax.fori_loop(0, J, body_b, (zero, zero))
        pull_s = jnp.sum(pacc_lo / safe_lo * valid_lo) + jnp.sum(
            pacc_hi / safe_hi * valid_hi)
        ntags = _bc(jnp.sum(valid_lo) + jnp.sum(valid_hi))
        scope_b.__exit__(None, None, None)

        scope_p = jax.named_scope("push")
        scope_p.__enter__()

        def push_pairs(src_m, src_v, n):
            def body_p(i, carry):
                acc_lo, acc_hi = carry
                sel = lane == i
                m_i = _bc(jnp.sum(jnp.where(sel, src_m, zero)))
                v_i = _bc(jnp.sum(jnp.where(sel, src_v, zero)))
                d_lo = m_i - mean_lo
                d_hi = m_i - mean_hi
                return (acc_lo + v_i * jnp.exp(-(d_lo * d_lo)) * valid_lo,
                        acc_hi + v_i * jnp.exp(-(d_hi * d_hi)) * valid_hi)

            return lax.fori_loop(0, n, body_p, (zero, zero))

        a_lo, a_hi = push_pairs(mean_lo, valid_lo, L)
        b_lo, b_hi = push_pairs(mean_hi, valid_hi, P - L)
        acc_lo = a_lo + b_lo
        acc_hi = a_hi + b_hi
        push_tot = _bc(jnp.sum(acc_lo) + jnp.sum(acc_hi)) - ntags
        denom = jnp.maximum(ntags * (ntags - one), one)
        push = 0.5 * push_tot / denom
        pull = _bc(pull_s) / jnp.maximum(ntags, one)
        scope_p.__exit__(None, None, None)

        with jax.named_scope("out"):
            oval_v[...] = jnp.where(
                lane == 0, pull, jnp.where(lane == 1, push, zero))
            pltpu.sync_copy(oval_v, out_hbm.at[wid])

    return aeloss


def kernel(input, input1):
    tags = input
    keypoints = input1
    B, N, D = tags.shape
    P, J = keypoints.shape[1], keypoints.shape[2]

    idx_t = keypoints[..., 0].transpose(0, 2, 1)
    flg_t = keypoints[..., 1].transpose(0, 2, 1)
    pad = ((0, 0), (0, JP - J), (0, PP - P))
    gidx = jnp.pad(idx_t, pad) + (jnp.arange(B, dtype=jnp.int32) * N)[:, None, None]
    flgp = jnp.pad(flg_t, pad)

    out = _build(B, N, P, J)(
        tags.reshape(B * N),
        gidx.reshape(B, GCH, GW),
        flgp.reshape(B, SLOTS),
    )
    return out[:, :2]


# speedup vs baseline: 1.1791x; 1.0304x over previous
import functools

import jax
import jax.numpy as jnp
from jax import lax
from jax.experimental import pallas as pl
from jax.experimental.pallas import tpu as pltpu
from jax.experimental.pallas import tpu_sc as plsc

L = 16
PP = 32
JP = 20
SLOTS = JP * PP
GCH = 16
GW = SLOTS // GCH


def _bc(s):
    return jnp.broadcast_to(s, (L,))


@functools.lru_cache(maxsize=None)
def _build(B, N, P, J):
    mesh = plsc.VectorSubcoreMesh(core_axis_name="c", subcore_axis_name="s")
    NC = 2

    @functools.partial(
        pl.kernel,
        mesh=mesh,
        out_type=jax.ShapeDtypeStruct((B, L), jnp.float32),
        compiler_params=pltpu.CompilerParams(needs_layout_passes=False),
        scratch_types=[
            pltpu.VMEM((2 * SLOTS,), jnp.int32),
            pltpu.VMEM((SLOTS,), jnp.float32),
            pltpu.VMEM((L,), jnp.float32),
            pltpu.SemaphoreType.DMA,
        ],
    )
    def aeloss(tags_hbm, pk_hbm, out_hbm, pk_v, val_v, oval_v, sem):
        wid = lax.axis_index("s") * NC + lax.axis_index("c")
        zero = jnp.zeros((L,), jnp.float32)
        one = jnp.full((L,), 1.0, jnp.float32)
        lane = lax.iota(jnp.int32, L)

        with jax.named_scope("stage"):
            pltpu.sync_copy(pk_hbm.at[wid, pl.ds(0, SLOTS)],
                            pk_v.at[pl.ds(0, SLOTS)])
        with jax.named_scope("fire"):
            copies = [
                pltpu.async_copy(
                    tags_hbm.at[pk_v.at[pl.ds(r * GW, GW)]],
                    val_v.at[pl.ds(r * GW, GW)], sem
                )
                for r in range(GCH)
            ]
        with jax.named_scope("stage2"):
            pltpu.sync_copy(pk_hbm.at[wid, pl.ds(SLOTS, SLOTS)],
                            pk_v.at[pl.ds(SLOTS, SLOTS)])
        with jax.named_scope("drain"):
            for cp in copies:
                cp.wait()

        scope_a = jax.named_scope("passA")
        scope_a.__enter__()

        def body_a(j, carry):
            cnt_lo, cnt_hi, sum_lo, sum_hi = carry
            base = j * PP
            f_lo = pk_v[pl.ds(SLOTS + base, L)]
            f_hi = pk_v[pl.ds(SLOTS + base + L, L)]
            v_lo = val_v[pl.ds(base, L)]
            v_hi = val_v[pl.ds(base + L, L)]
            w_lo = jnp.where(f_lo > 0, one, zero)
            w_hi = jnp.where(f_hi > 0, one, zero)
            return (cnt_lo + w_lo, cnt_hi + w_hi,
                    sum_lo + v_lo * w_lo, sum_hi + v_hi * w_hi)

        cnt_lo, cnt_hi, sum_lo, sum_hi = lax.fori_loop(
            0, J, body_a, (zero, zero, zero, zero))
        safe_lo = jnp.maximum(cnt_lo, one)
        safe_hi = jnp.maximum(cnt_hi, one)
        mean_lo = sum_lo / safe_lo
        mean_hi = sum_hi / safe_hi
        valid_lo = jnp.where(cnt_lo > 0, one, zero)
        valid_hi = jnp.where(cnt_hi > 0, one, zero)
        scope_a.__exit__(None, None, None)

        scope_b = jax.named_scope("passB")
        scope_b.__enter__()

        def body_b(j, carry):
            pacc_lo, pacc_hi = carry
            base = j * PP
            f_lo = pk_v[pl.ds(SLOTS + base, L)]
            f_hi = pk_v[pl.ds(SLOTS + base + L, L)]
            w_lo = jnp.where(f_lo > 0, one, zero)
            w_hi = jnp.where(f_hi > 0, one, zero)
            d_lo = val_v[pl.ds(base, L)] - mean_lo
            d_hi = val_v[pl.ds(base + L, L)] - mean_hi
            return (pacc_lo + d_lo * d_lo * w_lo, pacc_hi + d_hi * d_hi * w_hi)

        pacc_lo, pacc_hi = lax.fori_loop(0, J, body_b, (zero, zero))
        pull_s = jnp.sum(pacc_lo / safe_lo * valid_lo) + jnp.sum(
            pacc_hi / safe_hi * valid_hi)
        ntags = _bc(jnp.sum(valid_lo) + jnp.sum(valid_hi))
        scope_b.__exit__(None, None, None)

        scope_p = jax.named_scope("push")
        scope_p.__enter__()

        def push_pairs(src_m, src_v, n):
            def body_p(i, carry):
                acc_lo, acc_hi = carry
                sel = lane == i
                m_i = _bc(jnp.sum(jnp.where(sel, src_m, zero)))
                v_i = _bc(jnp.sum(jnp.where(sel, src_v, zero)))
                d_lo = m_i - mean_lo
                d_hi = m_i - mean_hi
                return (acc_lo + v_i * jnp.exp(-(d_lo * d_lo)) * valid_lo,
                        acc_hi + v_i * jnp.exp(-(d_hi * d_hi)) * valid_hi)

            return lax.fori_loop(0, n, body_p, (zero, zero))

        a_lo, a_hi = push_pairs(mean_lo, valid_lo, L)
        b_lo, b_hi = push_pairs(mean_hi, valid_hi, P - L)
        acc_lo = a_lo + b_lo
        acc_hi = a_hi + b_hi
        push_tot = _bc(jnp.sum(acc_lo) + jnp.sum(acc_hi)) - ntags
        denom = jnp.maximum(ntags * (ntags - one), one)
        push = 0.5 * push_tot / denom
        pull = _bc(pull_s) / jnp.maximum(ntags, one)
        scope_p.__exit__(None, None, None)

        with jax.named_scope("out"):
            oval_v[...] = jnp.where(
                lane == 0, pull, jnp.where(lane == 1, push, zero))
            pltpu.sync_copy(oval_v, out_hbm.at[wid])

    return aeloss


def kernel(input, input1):
    tags = input
    keypoints = input1
    B, N, D = tags.shape
    P, J = keypoints.shape[1], keypoints.shape[2]

    idx_t = keypoints[..., 0].transpose(0, 2, 1)
    flg_t = keypoints[..., 1].transpose(0, 2, 1)
    pad = ((0, 0), (0, JP - J), (0, PP - P))
    gidx = jnp.pad(idx_t, pad) + (jnp.arange(B, dtype=jnp.int32) * N)[:, None, None]
    flgp = jnp.pad(flg_t, pad)
    packed = jnp.concatenate(
        [gidx.reshape(B, SLOTS), flgp.reshape(B, SLOTS)], axis=1)

    out = _build(B, N, P, J)(tags.reshape(B * N), packed)
    return out[:, :2]
